# baseline (speedup 1.0000x reference)
import jax
import jax.numpy as jnp
from jax import lax
from jax.experimental import pallas as pl
from jax.experimental.pallas import tpu as pltpu
from jax.experimental.pallas import tpu_sc as plsc

N = 10000
E = 320000
D = 128
G = 512

NC = 2
NS = 16
NW = NC * NS
CHUNK = 64
PIECE = 40
NPIECE = E // (CHUNK * PIECE)
JMAX = -(-NPIECE // NW)
NP = 10240
ZROWS = NP // NS
ZSTEP = 40
NBUF = 4


def _sc_aggregate(x, edges3d):
    mesh = plsc.VectorSubcoreMesh(core_axis_name="c", subcore_axis_name="s")

    @pl.kernel(
        out_type=jax.ShapeDtypeStruct((NC, NP, D), jnp.float32),
        mesh=mesh,
        scratch_types=[
            pltpu.VMEM((PIECE, CHUNK), jnp.int32),
            pltpu.VMEM((PIECE, CHUNK), jnp.int32),
            pltpu.VMEM((NBUF, CHUNK, D), jnp.float32),
            pltpu.VMEM_SHARED((NP, D), jnp.float32),
        ] + [pltpu.SemaphoreType.DMA] * (2 * NBUF),
    )
    def agg_kernel(x_hbm, e_hbm, out_hbm, srcv, dstv, rows, acc, *sems):
        gsem = sems[:NBUF]
        ssem = sems[NBUF:]
        cid = lax.axis_index("c")
        sid = lax.axis_index("s")
        wid = cid * NS + sid

        @pl.loop(0, CHUNK)
        def _(i):
            @pl.loop(0, D, step=16)
            def _(j):
                rows.at[0, i, pl.ds(j, 16)][...] = jnp.zeros(
                    (16,), jnp.float32)

        @pl.loop(0, ZROWS, step=ZSTEP)
        def _(k):
            pltpu.sync_copy(rows.at[0, pl.ds(0, ZSTEP)],
                            acc.at[pl.ds(sid * ZROWS + k, ZSTEP)])

        plsc.subcore_barrier()

        for j in range(JMAX):
            @pl.when(j * NW + wid < NPIECE)
            def _():
                base = (j * NW + wid) * PIECE
                pltpu.async_copy(e_hbm.at[0, pl.ds(base, PIECE)], srcv,
                                 gsem[0])
                pltpu.async_copy(e_hbm.at[1, pl.ds(base, PIECE)], dstv,
                                 gsem[1])
                pltpu.make_async_copy(e_hbm.at[0, pl.ds(base, PIECE)], srcv,
                                      gsem[0]).wait()
                pltpu.make_async_copy(e_hbm.at[1, pl.ds(base, PIECE)], dstv,
                                      gsem[1]).wait()

                for b in range(NBUF):
                    pltpu.async_copy(x_hbm.at[srcv.at[b]], rows.at[b],
                                     gsem[b])

                @pl.loop(0, PIECE, step=NBUF)
                def _(r):
                    for b in range(NBUF):
                        pltpu.make_async_copy(
                            x_hbm.at[srcv.at[r + b]], rows.at[b],
                            gsem[b]).wait()
                        pltpu.async_copy(
                            rows.at[b], acc.at[dstv.at[r + b]], ssem[b],
                            add=True)
                    for b in range(NBUF):
                        @pl.when(r + NBUF + b < PIECE)
                        def _():
                            pltpu.make_async_copy(
                                rows.at[b], acc.at[dstv.at[r + b]],
                                ssem[b]).wait()
                            pltpu.async_copy(
                                x_hbm.at[srcv.at[r + NBUF + b]], rows.at[b],
                                gsem[b])

                for b in range(NBUF):
                    pltpu.make_async_copy(
                        rows.at[b], acc.at[dstv.at[PIECE - NBUF + b]],
                        ssem[b]).wait()

        plsc.subcore_barrier()

        pltpu.sync_copy(acc.at[pl.ds(sid * ZROWS, ZROWS)],
                        out_hbm.at[cid, pl.ds(sid * ZROWS, ZROWS)])

    return agg_kernel(x, edges3d)


NBLK = 5
BLK = N // NBLK


def _tc_body(x_ref, parts_ref, batch_ref, w1, b1, w2, b2, w3, b3, w4, b4,
             wl, bl, out_ref, sums, counts):
    i = pl.program_id(0)

    @pl.when(i == 0)
    def _():
        sums[...] = jnp.zeros_like(sums)
        counts[...] = jnp.zeros_like(counts)

    bf = jnp.bfloat16

    def dot16(a, w):
        return jnp.dot(a.astype(bf), w.astype(bf),
                       preferred_element_type=jnp.float32)

    h = x_ref[...] + parts_ref[0] + parts_ref[1]
    h = dot16(h, w1[...]) + b1[...]
    h = jnp.where(h > 0, h, 1.5 * h)
    h = jnp.maximum(dot16(h, w2[...]) + b2[...], 0.0)
    h = jnp.maximum(dot16(h, w3[...]) + b3[...], 0.0)
    h = dot16(h, w4[...]) + b4[...]

    seg = lax.broadcasted_iota(jnp.int32, (G, BLK), 0)
    eq = seg == batch_ref[0]
    sums[...] += dot16(eq.astype(jnp.float32), h)
    counts[...] += jnp.sum(jnp.where(eq, 1.0, 0.0), axis=1, keepdims=True)

    @pl.when(i == NBLK - 1)
    def _():
        mean = sums[...] / jnp.maximum(counts[...], 1.0)
        o = jnp.dot(mean, wl[...], preferred_element_type=jnp.float32) + bl[...]
        nrm = jnp.sqrt(jnp.sum(o * o, axis=1, keepdims=True))
        out_ref[...] = o / jnp.maximum(nrm, 1e-12)


def _tc_encode(x, parts, batch3d, W1, b1, W2, b2, W3, b3, W4, b4, Wl, bl):
    wspec = pl.BlockSpec((D, D), lambda i: (0, 0))
    bspec = pl.BlockSpec((1, D), lambda i: (0, 0))
    return pl.pallas_call(
        _tc_body,
        grid=(NBLK,),
        in_specs=[
            pl.BlockSpec((BLK, D), lambda i: (i, 0)),
            pl.BlockSpec((NC, BLK, D), lambda i: (0, i, 0)),
            pl.BlockSpec((1, 1, BLK), lambda i: (i, 0, 0)),
            wspec, bspec, wspec, bspec, wspec, bspec, wspec, bspec,
            wspec, bspec,
        ],
        out_specs=pl.BlockSpec((G, D), lambda i: (0, 0)),
        out_shape=jax.ShapeDtypeStruct((G, D), jnp.float32),
        scratch_shapes=[
            pltpu.VMEM((G, D), jnp.float32),
            pltpu.VMEM((G, 1), jnp.float32),
        ],
        compiler_params=pltpu.CompilerParams(
            dimension_semantics=("arbitrary",),
        ),
    )(x, parts, batch3d, W1, b1, W2, b2, W3, b3, W4, b4, Wl, bl)


def kernel(x, edge_index, batch, W1, b1, W2, b2, W3, b3, W4, b4, Wl, bl):
    edges3d = edge_index.reshape(2, E // CHUNK, CHUNK)
    parts = _sc_aggregate(x, edges3d)
    batch3d = batch.reshape(NBLK, 1, BLK)
    return _tc_encode(x, parts, batch3d,
                      W1, b1.reshape(1, D), W2, b2.reshape(1, D),
                      W3, b3.reshape(1, D), W4, b4.reshape(1, D),
                      Wl, bl.reshape(1, D))


# speedup vs baseline: 1.0130x; 1.0130x over previous
import jax
import jax.numpy as jnp
from jax import lax
from jax.experimental import pallas as pl
from jax.experimental.pallas import tpu as pltpu
from jax.experimental.pallas import tpu_sc as plsc

N = 10000
E = 320000
D = 128
G = 512

NC = 2
NS = 16
NW = NC * NS
CHUNK = 64
PIECE = 40
NPIECE = E // (CHUNK * PIECE)
JMAX = -(-NPIECE // NW)
NP = 10240
ZROWS = NP // NS
ZSTEP = 40
NBUF = 4


def _sc_aggregate(x, edges3d):
    mesh = plsc.VectorSubcoreMesh(core_axis_name="c", subcore_axis_name="s")

    @pl.kernel(
        out_type=jax.ShapeDtypeStruct((NC, NP, D), jnp.float32),
        mesh=mesh,
        scratch_types=[
            pltpu.VMEM((PIECE, CHUNK), jnp.int32),
            pltpu.VMEM((PIECE, CHUNK), jnp.int32),
            pltpu.VMEM((NBUF, CHUNK, D), jnp.float32),
            pltpu.VMEM_SHARED((NP, D), jnp.float32),
        ] + [pltpu.SemaphoreType.DMA] * (2 * NBUF),
    )
    def agg_kernel(x_hbm, e_hbm, out_hbm, srcv, dstv, rows, acc, *sems):
        gsem = sems[:NBUF]
        ssem = sems[NBUF:]
        cid = lax.axis_index("c")
        sid = lax.axis_index("s")
        wid = cid * NS + sid

        @pl.loop(0, CHUNK)
        def _(i):
            @pl.loop(0, D, step=16)
            def _(j):
                rows.at[0, i, pl.ds(j, 16)][...] = jnp.zeros(
                    (16,), jnp.float32)

        @pl.loop(0, ZROWS, step=ZSTEP)
        def _(k):
            pltpu.sync_copy(rows.at[0, pl.ds(0, ZSTEP)],
                            acc.at[pl.ds(sid * ZROWS + k, ZSTEP)])

        plsc.subcore_barrier()

        for j in range(JMAX):
            @pl.when(j * NW + wid < NPIECE)
            def _():
                base = (j * NW + wid) * PIECE
                pltpu.async_copy(e_hbm.at[0, pl.ds(base, PIECE)], srcv,
                                 gsem[0])
                pltpu.async_copy(e_hbm.at[1, pl.ds(base, PIECE)], dstv,
                                 gsem[1])
                pltpu.make_async_copy(e_hbm.at[0, pl.ds(base, PIECE)], srcv,
                                      gsem[0]).wait()
                pltpu.make_async_copy(e_hbm.at[1, pl.ds(base, PIECE)], dstv,
                                      gsem[1]).wait()

                for b in range(NBUF):
                    pltpu.async_copy(x_hbm.at[srcv.at[b]], rows.at[b],
                                     gsem[b])

                @pl.loop(0, PIECE, step=NBUF)
                def _(r):
                    for b in range(NBUF):
                        pltpu.make_async_copy(
                            x_hbm.at[srcv.at[r + b]], rows.at[b],
                            gsem[b]).wait()
                        pltpu.async_copy(
                            rows.at[b], acc.at[dstv.at[r + b]], ssem[b],
                            add=True)
                    for b in range(NBUF):
                        @pl.when(r + NBUF + b < PIECE)
                        def _():
                            pltpu.make_async_copy(
                                rows.at[b], acc.at[dstv.at[r + b]],
                                ssem[b]).wait()
                            pltpu.async_copy(
                                x_hbm.at[srcv.at[r + NBUF + b]], rows.at[b],
                                gsem[b])

                for b in range(NBUF):
                    pltpu.make_async_copy(
                        rows.at[b], acc.at[dstv.at[PIECE - NBUF + b]],
                        ssem[b]).wait()

        plsc.subcore_barrier()

        pltpu.sync_copy(acc.at[pl.ds(sid * ZROWS, ZROWS)],
                        out_hbm.at[cid, pl.ds(sid * ZROWS, ZROWS)])

    return agg_kernel(x, edges3d)


NBLK = 5
BLK = N // NBLK


def _tc_body(x_ref, parts_ref, batch_ref, w1, b1, w2, b2, w3, b3, w4, b4,
             wl, bl, out_ref, sums, counts):
    i = pl.program_id(0)

    @pl.when(i == 0)
    def _():
        sums[...] = jnp.zeros_like(sums)
        counts[...] = jnp.zeros_like(counts)

    h = x_ref[...] + parts_ref[0] + parts_ref[1]
    h = jnp.dot(h, w1[...], preferred_element_type=jnp.float32) + b1[...]
    h = jnp.where(h > 0, h, 1.5 * h)
    h = jnp.dot(h, w2[...], preferred_element_type=jnp.float32) + b2[...]
    h = jnp.maximum(h, 0.0)
    h = jnp.dot(h, w3[...], preferred_element_type=jnp.float32) + b3[...]
    h = jnp.maximum(h, 0.0)
    h = jnp.dot(h, w4[...], preferred_element_type=jnp.float32) + b4[...]

    seg = lax.broadcasted_iota(jnp.int32, (G, BLK), 0)
    onehot = (seg == batch_ref[0]).astype(jnp.float32)
    sums[...] += jnp.dot(onehot, h, preferred_element_type=jnp.float32)
    counts[...] += jnp.sum(onehot, axis=1, keepdims=True)

    @pl.when(i == NBLK - 1)
    def _():
        mean = sums[...] / jnp.maximum(counts[...], 1.0)
        o = jnp.dot(mean, wl[...], preferred_element_type=jnp.float32) + bl[...]
        nrm = jnp.sqrt(jnp.sum(o * o, axis=1, keepdims=True))
        out_ref[...] = o / jnp.maximum(nrm, 1e-12)


def _tc_encode(x, parts, batch3d, W1, b1, W2, b2, W3, b3, W4, b4, Wl, bl):
    wspec = pl.BlockSpec((D, D), lambda i: (0, 0))
    bspec = pl.BlockSpec((1, D), lambda i: (0, 0))
    return pl.pallas_call(
        _tc_body,
        grid=(NBLK,),
        in_specs=[
            pl.BlockSpec((BLK, D), lambda i: (i, 0)),
            pl.BlockSpec((NC, BLK, D), lambda i: (0, i, 0)),
            pl.BlockSpec((1, 1, BLK), lambda i: (i, 0, 0)),
            wspec, bspec, wspec, bspec, wspec, bspec, wspec, bspec,
            wspec, bspec,
        ],
        out_specs=pl.BlockSpec((G, D), lambda i: (0, 0)),
        out_shape=jax.ShapeDtypeStruct((G, D), jnp.float32),
        scratch_shapes=[
            pltpu.VMEM((G, D), jnp.float32),
            pltpu.VMEM((G, 1), jnp.float32),
        ],
        compiler_params=pltpu.CompilerParams(
            dimension_semantics=("arbitrary",),
        ),
    )(x, parts, batch3d, W1, b1, W2, b2, W3, b3, W4, b4, Wl, bl)


def kernel(x, edge_index, batch, W1, b1, W2, b2, W3, b3, W4, b4, Wl, bl):
    edges3d = edge_index.reshape(2, E // CHUNK, CHUNK)
    parts = _sc_aggregate(x, edges3d)
    batch3d = batch.reshape(NBLK, 1, BLK)
    return _tc_encode(x, parts, batch3d,
                      W1, b1.reshape(1, D), W2, b2.reshape(1, D),
                      W3, b3.reshape(1, D), W4, b4.reshape(1, D),
                      Wl, bl.reshape(1, D))
